# baseline (speedup 1.0000x reference)
import functools

import jax
import jax.numpy as jnp
from jax.experimental import pallas as pl
from jax.experimental.pallas import tpu as pltpu

_BM = 512
_DEPTH = 8


def _fused_kernel(x_hbm, wt_ref, pns_ref, out_ref, buf, sem):
    i = pl.program_id(0)
    n = pl.num_programs(0)

    @pl.when(i == 0)
    def _prologue():
        for dd in range(_DEPTH):
            pltpu.make_async_copy(
                x_hbm.at[pl.ds(dd * _BM, _BM), :], buf.at[dd], sem.at[dd]
            ).start()

    slot = jax.lax.rem(i, _DEPTH)
    pltpu.make_async_copy(
        x_hbm.at[pl.ds(i * _BM, _BM), :], buf.at[slot], sem.at[slot]
    ).wait()

    t = jnp.dot(buf[slot], wt_ref[...], preferred_element_type=jnp.float32)

    @pl.when(i + _DEPTH < n)
    def _issue_next():
        nxt = i + _DEPTH
        pltpu.make_async_copy(
            x_hbm.at[pl.ds(nxt * _BM, _BM), :], buf.at[slot], sem.at[slot]
        ).start()

    u = jnp.tanh(t.T)
    s = jnp.sum(u * u, axis=0, keepdims=True)
    r = jax.lax.rsqrt(jnp.maximum(s, 1e-12))
    d = jnp.dot(pns_ref[...], u, preferred_element_type=jnp.float32)
    e = jnp.exp(d * r)
    out_ref[...] = e / jnp.sum(e, axis=0, keepdims=True)


@functools.partial(jax.jit, static_argnames=())
def _run(x2d, wt, pns):
    n_rows, dk = x2d.shape
    grid = (n_rows // _BM,)
    return pl.pallas_call(
        _fused_kernel,
        grid=grid,
        in_specs=[
            pl.BlockSpec(memory_space=pltpu.MemorySpace.HBM),
            pl.BlockSpec(wt.shape, lambda i: (0, 0)),
            pl.BlockSpec(pns.shape, lambda i: (0, 0)),
        ],
        out_specs=pl.BlockSpec((8, _BM), lambda i: (0, i)),
        out_shape=jax.ShapeDtypeStruct((8, n_rows), jnp.float32),
        scratch_shapes=[
            pltpu.VMEM((_DEPTH, _BM, 1024), jnp.float32),
            pltpu.SemaphoreType.DMA((_DEPTH,)),
        ],
        compiler_params=pltpu.CompilerParams(
            dimension_semantics=("arbitrary",),
        ),
    )(x2d, wt, pns)


def kernel(x, W, prototypes, hamming_scale):
    b, s, d = x.shape
    k = prototypes.shape[0]
    x2d = x.reshape(b * s, d)
    pn = prototypes / jnp.maximum(
        jnp.linalg.norm(prototypes, axis=-1, keepdims=True), 1e-12
    )
    pns = (3.0 * jnp.asarray(hamming_scale, jnp.float32)) * pn
    out = _run(x2d, W.T, pns)
    return out.T.reshape(b, s, k)
